# baseline (speedup 1.0000x reference)
import functools
import math

import jax
import jax.numpy as jnp
from jax import lax
from jax.experimental import pallas as pl
from jax.experimental.pallas import tpu as pltpu
from jax.experimental.pallas import tpu_sc as plsc

_VOCAB = 100000
_PAD = 0
_SMOOTH = 0.1
_CONF = 1.0 - _SMOOTH
_FILL = _SMOOTH / (_VOCAB - 2)
_C = _CONF * math.log(_CONF) + _SMOOTH * math.log(_FILL)

_BLOCK_COLS = 4096
_SPLIT = 768
_NW = 32
_GCH = 5760
_N_GCH = 17
_LCH = 2048
_FULL_COLS = 97920 + 2048
_TAIL = _VOCAB - 99840


def _tc_kernel(y_ref, t_ref, out_ref, *, block_cols, vocab):
    k = pl.program_id(0)
    base = k * block_cols
    col = base + jax.lax.broadcasted_iota(jnp.int32, (1, block_cols), 1)
    x = jnp.where(col < vocab, y_ref[...], 0.0)

    t = t_ref[...]
    valid = (t != _PAD).astype(jnp.float32)

    s_part = jnp.sum(x, axis=1, keepdims=True)
    s_valid = jnp.sum(s_part * valid, keepdims=True)

    g = jnp.where(col == t, x, 0.0)
    g_sum = jnp.sum(jnp.sum(g, axis=1, keepdims=True) * valid, keepdims=True)

    contrib = -_FILL * s_valid - (_CONF - _FILL) * g_sum

    @pl.when(k == 0)
    def _():
        z_sum = jnp.sum(x[:, 0:1] * valid, keepdims=True)
        n_valid = jnp.sum(valid, keepdims=True)
        out_ref[...] = contrib + _FILL * z_sum + n_valid * _C

    @pl.when(k != 0)
    def _():
        out_ref[...] += contrib


def _make_sc_kernel(n_rows, vocab, split):
    rpw = n_rows // _NW
    assert rpw == 8, "each subcore owns exactly one (8,128) row group"
    mesh = plsc.VectorSubcoreMesh(core_axis_name="c", subcore_axis_name="s")

    @functools.partial(
        pl.kernel, mesh=mesh,
        out_type=jax.ShapeDtypeStruct((_NW, 16), jnp.float32),
        scratch_types=[pltpu.VMEM((8, _GCH), jnp.float32),
                       pltpu.VMEM((8, _GCH), jnp.float32),
                       pltpu.VMEM((8, _LCH), jnp.float32),
                       pltpu.VMEM((32,), jnp.int32),
                       pltpu.VMEM((128,), jnp.float32),
                       pltpu.VMEM((32,), jnp.float32),
                       pltpu.VMEM((16,), jnp.float32),
                       pltpu.SemaphoreType.DMA,
                       pltpu.SemaphoreType.DMA,
                       pltpu.SemaphoreType.DMA],
    )
    def sc_kernel(y_hbm, t_hbm, out_hbm, buf0, buf1, buf2, tbuf, gbuf,
                  tailbuf, ob, sem0, sem1, sem2):
        wid = lax.axis_index("s") * 2 + lax.axis_index("c")
        base = split + wid * rpw
        tslot = (base // 16) * 16
        pltpu.sync_copy(t_hbm.at[pl.ds(tslot, 16)], tbuf.at[pl.ds(0, 16)])
        lid = lax.iota(jnp.int32, 16)
        bufs = (buf0, buf1)
        sems = (sem0, sem1)

        h2 = pltpu.async_copy(
            y_hbm.at[pl.ds(base, 8), pl.ds(_N_GCH * _GCH, _LCH)], buf2, sem2)

        def accum_group(b, width, accs):
            def inner(j, a):
                o = j * 16
                return tuple(a[r] + b[r, pl.ds(o, 16)] for r in range(8))

            return lax.fori_loop(0, width // 16, inner, accs)

        h = pltpu.async_copy(
            y_hbm.at[pl.ds(base, 8), pl.ds(0, _GCH)], bufs[0], sems[0])
        accs = tuple(jnp.zeros((16,), jnp.float32) for _ in range(8))
        for c in range(_N_GCH):
            if c + 1 < _N_GCH:
                h_next = pltpu.async_copy(
                    y_hbm.at[pl.ds(base, 8), pl.ds((c + 1) * _GCH, _GCH)],
                    bufs[(c + 1) % 2], sems[(c + 1) % 2])
            h.wait()
            accs = accum_group(bufs[c % 2], _GCH, accs)
            if c + 1 < _N_GCH:
                h = h_next
        h2.wait()
        accs = accum_group(buf2, _LCH, accs)

        contrib = jnp.zeros((16,), jnp.float32)
        for r in range(8):
            row = base + r
            t = tbuf[pl.ds(base - tslot + r, 16)][0]
            accv = accs[r]

            pltpu.sync_copy(y_hbm.at[row, pl.ds(_FULL_COLS, _TAIL - 128)],
                            tailbuf.at[pl.ds(0, _TAIL - 128)])
            for j in range((_TAIL - 128) // 16):
                accv = accv + tailbuf[pl.ds(j * 16, 16)]

            off = (t // 128) * 128
            pltpu.sync_copy(y_hbm.at[row, pl.ds(off, 128)], gbuf)
            lane = t - off
            gacc = jnp.zeros((16,), jnp.float32)
            for j in range(8):
                v = gbuf[pl.ds(j * 16, 16)]
                gacc = gacc + jnp.where(lid + (j * 16) == lane, v, 0.0)

            pltpu.sync_copy(y_hbm.at[row, pl.ds(0, 16)], ob)
            zvec = jnp.where(lid == 0, ob[...], 0.0)

            rowv = (-(_CONF - _FILL) * gacc - _FILL * (accv - zvec)
                    + jnp.where(lid == 0, _C, 0.0))
            contrib = contrib + jnp.where(t != _PAD, rowv, 0.0)

        ob[...] = contrib
        pltpu.sync_copy(ob, out_hbm.at[wid])

    return sc_kernel


def kernel(yhat, target):
    n, vocab = yhat.shape
    t = target.astype(jnp.int32)
    t2 = t.reshape(n, 1)

    sc_out = _make_sc_kernel(n - _SPLIT, vocab, _SPLIT)(yhat, t)

    n_blocks = pl.cdiv(vocab, _BLOCK_COLS)
    tc_out = pl.pallas_call(
        functools.partial(_tc_kernel, block_cols=_BLOCK_COLS, vocab=vocab),
        grid=(n_blocks,),
        in_specs=[
            pl.BlockSpec((_SPLIT, _BLOCK_COLS), lambda k: (0, k)),
            pl.BlockSpec((_SPLIT, 1), lambda k: (0, 0)),
        ],
        out_specs=pl.BlockSpec((1, 1), lambda k: (0, 0)),
        out_shape=jax.ShapeDtypeStruct((1, 1), jnp.float32),
    )(yhat, t2)

    return tc_out[0, 0] + jnp.sum(sc_out)


# speedup vs baseline: 1.0519x; 1.0519x over previous
import functools
import math

import jax
import jax.numpy as jnp
from jax.experimental import pallas as pl

_VOCAB = 100000
_PAD = 0
_SMOOTH = 0.1
_CONF = 1.0 - _SMOOTH
_FILL = _SMOOTH / (_VOCAB - 2)
_C = _CONF * math.log(_CONF) + _SMOOTH * math.log(_FILL)

_BLOCK_COLS = 4608


def _ls_kernel(y_ref, t_ref, out_ref, *, block_cols, vocab):
    k = pl.program_id(0)
    base = k * block_cols
    col = base + jax.lax.broadcasted_iota(jnp.int32, (1, block_cols), 1)
    x = jnp.where(col < vocab, y_ref[...], 0.0)

    t = t_ref[...]
    valid = (t != _PAD).astype(jnp.float32)

    s_part = jnp.sum(x, axis=1, keepdims=True)
    s_valid = jnp.sum(s_part * valid, keepdims=True)

    g = jnp.where(col == t, x, 0.0)
    g_sum = jnp.sum(jnp.sum(g, axis=1, keepdims=True) * valid, keepdims=True)

    contrib = -_FILL * s_valid - (_CONF - _FILL) * g_sum

    @pl.when(k == 0)
    def _():
        z_sum = jnp.sum(x[:, 0:1] * valid, keepdims=True)
        n_valid = jnp.sum(valid, keepdims=True)
        out_ref[...] = contrib + _FILL * z_sum + n_valid * _C

    @pl.when(k != 0)
    def _():
        out_ref[...] += contrib


def kernel(yhat, target):
    n, vocab = yhat.shape
    t2 = target.astype(jnp.int32).reshape(n, 1)
    grid = pl.cdiv(vocab, _BLOCK_COLS)
    out = pl.pallas_call(
        functools.partial(_ls_kernel, block_cols=_BLOCK_COLS, vocab=vocab),
        grid=(grid,),
        in_specs=[
            pl.BlockSpec((n, _BLOCK_COLS), lambda k: (0, k)),
            pl.BlockSpec((n, 1), lambda k: (0, 0)),
        ],
        out_specs=pl.BlockSpec((1, 1), lambda k: (0, 0)),
        out_shape=jax.ShapeDtypeStruct((1, 1), jnp.float32),
    )(yhat, t2)
    return out[0, 0]
